# baseline (speedup 1.0000x reference)
import functools

import jax
import jax.numpy as jnp
from jax import lax
from jax.experimental import pallas as pl
from jax.experimental.pallas import tpu as pltpu
from jax.experimental.pallas import tpu_sc as plsc

F = 32
L = 16
CH = 128


def kernel(user_ids, item_ids, user_emb, item_emb, user_bias, item_bias, global_bias):
    B = user_ids.shape[0]
    info = plsc.get_sparse_core_info()
    NC, NS = info.num_cores, info.num_subcores
    NW = NC * NS
    b_per_w = B // NW
    n_ch = b_per_w // CH

    uid2d = jnp.asarray(user_ids, jnp.int32).reshape(B // CH, CH)
    iid2d = jnp.asarray(item_ids, jnp.int32).reshape(B // CH, CH)
    gb16 = jnp.broadcast_to(jnp.asarray(global_bias, jnp.float32).reshape(1), (L,))

    mesh = plsc.VectorSubcoreMesh(core_axis_name="c", subcore_axis_name="s")

    @functools.partial(
        pl.kernel,
        mesh=mesh,
        compiler_params=pltpu.CompilerParams(
            needs_layout_passes=False, use_tc_tiling_on_sc=False),
        out_type=jax.ShapeDtypeStruct((B,), jnp.float32),
        scratch_types=[
            pltpu.VMEM((n_ch, CH), jnp.int32),
            pltpu.VMEM((n_ch, CH), jnp.int32),
            pltpu.VMEM((b_per_w, F), jnp.float32),
            pltpu.VMEM((b_per_w, F), jnp.float32),
            pltpu.VMEM((b_per_w,), jnp.float32),
            pltpu.VMEM((b_per_w,), jnp.float32),
            pltpu.VMEM((b_per_w,), jnp.float32),
            pltpu.VMEM((L,), jnp.float32),
            pltpu.SemaphoreType.DMA,
        ],
    )
    def _k(uid_hbm, iid_hbm, uemb_hbm, iemb_hbm, ubias_hbm, ibias_hbm,
           gb_hbm, out_hbm, uidx, iidx, urows, vrows, bu, bi, outb, gbv, sem):
        wid = lax.axis_index("s") * NC + lax.axis_index("c")
        row0 = wid * n_ch
        base = wid * b_per_w

        pltpu.sync_copy(uid_hbm.at[pl.ds(row0, n_ch)], uidx)
        pltpu.sync_copy(iid_hbm.at[pl.ds(row0, n_ch)], iidx)
        pltpu.sync_copy(gb_hbm, gbv)

        copies = []
        for j in range(n_ch):
            s = pl.ds(j * CH, CH)
            copies.append(pltpu.async_copy(uemb_hbm.at[uidx.at[j]], urows.at[s], sem))
            copies.append(pltpu.async_copy(iemb_hbm.at[iidx.at[j]], vrows.at[s], sem))
            copies.append(pltpu.async_copy(ubias_hbm.at[uidx.at[j]], bu.at[s], sem))
            copies.append(pltpu.async_copy(ibias_hbm.at[iidx.at[j]], bi.at[s], sem))
        for c in copies:
            c.wait()

        gv = gbv[...]
        lane = lax.iota(jnp.int32, L)

        def blk_body(r, _):
            rows = lane + r * L
            acc = gv
            for f in range(F):
                col = jnp.full((L,), f, jnp.int32)
                uc = plsc.load_gather(urows, [rows, col])
                vc = plsc.load_gather(vrows, [rows, col])
                acc = acc + uc * vc
            s = pl.ds(r * L, L)
            outb[s] = acc + bu[s] + bi[s]
            return 0

        lax.fori_loop(0, b_per_w // L, blk_body, 0, unroll=2)

        pltpu.sync_copy(outb, out_hbm.at[pl.ds(base, b_per_w)])

    return _k(uid2d, iid2d, user_emb, item_emb, user_bias, item_bias, gb16)


# speedup vs baseline: 4.0443x; 4.0443x over previous
import functools

import jax
import jax.numpy as jnp
from jax import lax
from jax.experimental import pallas as pl
from jax.experimental.pallas import tpu as pltpu
from jax.experimental.pallas import tpu_sc as plsc

F = 32
L = 16
CH = 128
SLAB = 8
NSLAB = F // SLAB


def _slab_flats(emb, n_main):
    flats = []
    nt = n_main // CH
    for t in range(NSLAB):
        a = lax.slice(emb, (0, SLAB * t), (n_main, SLAB * (t + 1)))
        flats.append(a.reshape(nt, CH, SLAB).transpose(0, 2, 1).reshape(-1))
    return flats


def kernel(user_ids, item_ids, user_emb, item_emb, user_bias, item_bias, global_bias):
    B = user_ids.shape[0]
    N = user_emb.shape[0]
    info = plsc.get_sparse_core_info()
    NC, NS = info.num_cores, info.num_subcores
    NW = NC * NS
    b_per_w = B // NW
    n_ch = b_per_w // CH
    n_main = (N // CH) * CH
    n_rem = N - n_main
    n_rows = SLAB * n_ch

    uid2d = jnp.asarray(user_ids, jnp.int32).reshape(B // CH, CH)
    iid2d = jnp.asarray(item_ids, jnp.int32).reshape(B // CH, CH)
    gb16 = jnp.broadcast_to(jnp.asarray(global_bias, jnp.float32).reshape(1), (L,))
    uflats = _slab_flats(user_emb, n_main)
    iflats = _slab_flats(item_emb, n_main)
    rem_u = lax.slice(user_emb, (n_main, 0), (N, F))
    rem_i = lax.slice(item_emb, (n_main, 0), (N, F))

    mesh = plsc.VectorSubcoreMesh(core_axis_name="c", subcore_axis_name="s")

    @functools.partial(
        pl.kernel,
        mesh=mesh,
        compiler_params=pltpu.CompilerParams(
            needs_layout_passes=False, use_tc_tiling_on_sc=False),
        out_type=jax.ShapeDtypeStruct((B,), jnp.float32),
        scratch_types=[
            pltpu.VMEM((n_ch, CH), jnp.int32),
            pltpu.VMEM((n_ch, CH), jnp.int32),
            pltpu.VMEM((n_rows, CH), jnp.int32),
            pltpu.VMEM((n_rows, CH), jnp.int32),
            pltpu.VMEM((NSLAB, n_rows, CH), jnp.float32),
            pltpu.VMEM((NSLAB, n_rows, CH), jnp.float32),
            pltpu.VMEM((n_ch, CH), jnp.float32),
            pltpu.VMEM((n_ch, CH), jnp.float32),
            pltpu.VMEM((n_rem, F), jnp.float32),
            pltpu.VMEM((n_rem, F), jnp.float32),
            pltpu.VMEM((b_per_w,), jnp.float32),
            pltpu.VMEM((L,), jnp.float32),
            pltpu.SemaphoreType.DMA,
        ],
    )
    def _k(uid_hbm, iid_hbm,
           uf0, uf1, uf2, uf3, if0, if1, if2, if3,
           remu_hbm, remi_hbm, ubias_hbm, ibias_hbm, gb_hbm,
           out_hbm,
           uidx, iidx, uoffs, ioffs, ubuf, vbuf, bu, bi,
           remu, remi, outb, gbv, sem):
        ufs = (uf0, uf1, uf2, uf3)
        ifs = (if0, if1, if2, if3)
        wid = lax.axis_index("s") * NC + lax.axis_index("c")
        row0 = wid * n_ch
        base = wid * b_per_w

        pltpu.sync_copy(uid_hbm.at[pl.ds(row0, n_ch)], uidx)
        pltpu.sync_copy(iid_hbm.at[pl.ds(row0, n_ch)], iidx)
        pltpu.sync_copy(gb_hbm, gbv)
        pltpu.sync_copy(remu_hbm, remu)
        pltpu.sync_copy(remi_hbm, remi)

        def offs_body(m, _):
            def sub_body(s, _):
                cols = pl.ds(s * L, L)
                for ids_ref, offs_ref in ((uidx, uoffs), (iidx, ioffs)):
                    r = ids_ref[m, cols]
                    rc = jnp.minimum(r, n_main - 1)
                    b0 = ((rc >> 7) << 10) + (rc & 127)
                    for i1 in range(SLAB):
                        offs_ref[i1 * n_ch + m, cols] = b0 + i1 * CH
                return 0
            lax.fori_loop(0, CH // L, sub_body, 0)
            return 0

        lax.fori_loop(0, n_ch, offs_body, 0)

        copies = []
        for t in range(NSLAB):
            for j in range(n_rows):
                copies.append(pltpu.async_copy(
                    ufs[t].at[uoffs.at[j]], ubuf.at[t, j], sem))
                copies.append(pltpu.async_copy(
                    ifs[t].at[ioffs.at[j]], vbuf.at[t, j], sem))
        for m in range(n_ch):
            copies.append(pltpu.async_copy(
                ubias_hbm.at[uidx.at[m]], bu.at[m], sem))
            copies.append(pltpu.async_copy(
                ibias_hbm.at[iidx.at[m]], bi.at[m], sem))
        for c in copies:
            c.wait()

        gv = gbv[...]

        def fix_body(c16, _):
            m = c16 // (CH // L)
            cols = pl.ds((c16 % (CH // L)) * L, L)
            for ids_ref, buf, rem in ((uidx, ubuf, remu), (iidx, vbuf, remi)):
                r = ids_ref[m, cols]
                rmax = lax.reduce_max(r, axes=(0,))

                @pl.when(rmax >= n_main)
                def _():
                    tail = r >= n_main
                    rr = jnp.maximum(r - n_main, 0)
                    for t in range(NSLAB):
                        for i1 in range(SLAB):
                            cur = buf[t, i1 * n_ch + m, cols]
                            g = plsc.load_gather(
                                rem, [rr, jnp.full((L,), SLAB * t + i1, jnp.int32)])
                            buf[t, i1 * n_ch + m, cols] = jnp.where(tail, g, cur)
                return 0
            return 0

        lax.fori_loop(0, b_per_w // L, fix_body, 0)

        def acc_body(c16, _):
            m = c16 // (CH // L)
            cols = pl.ds((c16 % (CH // L)) * L, L)
            acc = gv + bu[m, cols] + bi[m, cols]
            for t in range(NSLAB):
                for i1 in range(SLAB):
                    j = i1 * n_ch + m
                    acc = acc + ubuf[t, j, cols] * vbuf[t, j, cols]
            outb[pl.ds(c16 * L, L)] = acc
            return 0

        lax.fori_loop(0, b_per_w // L, acc_body, 0)

        pltpu.sync_copy(outb, out_hbm.at[pl.ds(base, b_per_w)])

    return _k(uid2d, iid2d, *uflats, *iflats, rem_u, rem_i,
              user_bias, item_bias, gb16)
